# baseline (speedup 1.0000x reference)
import functools

import jax
import jax.numpy as jnp
from jax import lax
from jax.experimental import pallas as pl
from jax.experimental.pallas import tpu as pltpu
from jax.experimental.pallas import tpu_sc as plsc

_NC = 2
_NS = 16
_NW = _NC * _NS
_LANES = 16
_CHUNK = 128


def _neg_sqrt(s):
    bits = plsc.bitcast(s, jnp.int32)
    r = plsc.bitcast(jnp.int32(0x5F3759DF) - (bits >> 1), jnp.float32)
    half_s = 0.5 * s
    for _ in range(3):
        r = r * (1.5 - half_s * r * r)
    return -(s * r)


def _make_sc_kernel(batch, dim, chunks):
    b_per_w = chunks * _CHUNK
    groups = b_per_w // _LANES

    mesh = plsc.VectorSubcoreMesh(core_axis_name="c", subcore_axis_name="s")

    @functools.partial(
        pl.kernel,
        mesh=mesh,
        compiler_params=pltpu.CompilerParams(
            needs_layout_passes=False, use_tc_tiling_on_sc=False),
        out_type=jax.ShapeDtypeStruct((batch,), jnp.float32),
        scratch_types=[
            pltpu.VMEM((chunks, _CHUNK), jnp.int32),
            pltpu.VMEM((chunks, _CHUNK), jnp.int32),
            pltpu.VMEM((chunks, _CHUNK), jnp.int32),
            pltpu.VMEM((b_per_w, dim), jnp.float32),
            pltpu.VMEM((b_per_w, dim), jnp.float32),
            pltpu.VMEM((b_per_w, dim), jnp.float32),
            pltpu.VMEM((b_per_w,), jnp.float32),
            pltpu.SemaphoreType.DMA,
        ],
    )
    def sc_kernel(heads_hbm, rels_hbm, tails_hbm, ent_hbm, rel_hbm, out_hbm,
                  idx_h, idx_r, idx_t, hrows, rrows, trows, outv, sem):
        wid = lax.axis_index("s") * _NC + lax.axis_index("c")

        pltpu.sync_copy(heads_hbm.at[wid], idx_h)
        pltpu.sync_copy(rels_hbm.at[wid], idx_r)
        pltpu.sync_copy(tails_hbm.at[wid], idx_t)

        copies = []
        for j in range(chunks):
            dst = pl.ds(j * _CHUNK, _CHUNK)
            copies.append(pltpu.async_copy(ent_hbm.at[idx_h.at[j]], hrows.at[dst], sem))
            copies.append(pltpu.async_copy(rel_hbm.at[idx_r.at[j]], rrows.at[dst], sem))
            copies.append(pltpu.async_copy(ent_hbm.at[idx_t.at[j]], trows.at[dst], sem))
        for c in copies:
            c.wait()

        vregs_per_row = dim // _LANES
        lane = lax.broadcasted_iota(jnp.int32, (_LANES,), 0)

        def group_body(g, carry):
            base = g * _LANES
            s = jnp.zeros((_LANES,), jnp.float32)
            for l in range(_LANES):
                row = base + l
                acc = None
                for j in range(vregs_per_row):
                    sl = pl.ds(j * _LANES, _LANES)
                    d = hrows[row, sl] + rrows[row, sl] - trows[row, sl]
                    sq = d * d
                    acc = sq if acc is None else acc + sq
                s = jnp.where(lane == l, jnp.sum(acc), s)
            outv[pl.ds(base, _LANES)] = _neg_sqrt(s)
            return carry

        lax.fori_loop(0, groups, group_body, 0)

        pltpu.sync_copy(outv, out_hbm.at[pl.ds(wid * b_per_w, b_per_w)])

    return sc_kernel


def kernel(heads, relations, tails, entity_emb, relation_emb):
    batch = heads.shape[0]
    dim = entity_emb.shape[1]
    chunks = batch // (_NW * _CHUNK)
    h3 = heads.reshape(_NW, chunks, _CHUNK)
    r3 = relations.reshape(_NW, chunks, _CHUNK)
    t3 = tails.reshape(_NW, chunks, _CHUNK)
    out = _make_sc_kernel(batch, dim, chunks)(h3, r3, t3, entity_emb, relation_emb)
    return out.reshape(batch, 1)
